# baseline (speedup 1.0000x reference)
import functools

import numpy as np
import jax
import jax.numpy as jnp
from jax import lax
from jax.experimental import pallas as pl
from jax.experimental.pallas import tpu as pltpu, tpu_sc as plsc

_PROB = 0.25

_DROPPED_B16 = (7, 6, 3, 2)


@functools.lru_cache(maxsize=None)
def _dropped_ids(b):
    num = 1 if b == 1 else int(b * _PROB)
    if b == 16:
        return _DROPPED_B16[:num]
    with jax.ensure_compile_time_eval(), jax.default_device(jax.devices("cpu")[0]):
        perm = np.asarray(jax.random.permutation(jax.random.key(1), b))
    return tuple(int(x) for x in perm[:num])


def _sc_part(emb0, s, dropped):
    b, t, c, d = emb0.shape
    info = plsc.get_sparse_core_info()
    nw = info.num_cores * info.num_subcores
    wpb = nw // s
    tn = t // wpb
    ch = 32
    nch = tn // ch
    mesh = plsc.VectorSubcoreMesh(core_axis_name="c", subcore_axis_name="s")

    zeros = jnp.zeros((ch, c, d), emb0.dtype)

    @functools.partial(
        pl.kernel,
        out_type=jax.ShapeDtypeStruct((s, t, c, d), emb0.dtype),
        mesh=mesh,
        scratch_types=[
            pltpu.VMEM((ch, c, d), emb0.dtype),
            pltpu.VMEM((ch, c, d), emb0.dtype),
            pltpu.VMEM((ch, c, d), emb0.dtype),
            pltpu.VMEM((ch, c, d), emb0.dtype),
            pltpu.VMEM((ch, c, d), emb0.dtype),
            pltpu.VMEM((ch, c, d), emb0.dtype),
            pltpu.SemaphoreType.DMA,
            pltpu.SemaphoreType.DMA,
            pltpu.SemaphoreType.DMA,
            pltpu.SemaphoreType.DMA,
            pltpu.SemaphoreType.DMA,
            pltpu.SemaphoreType.DMA,
            pltpu.SemaphoreType.DMA,
            pltpu.SemaphoreType.DMA,
            pltpu.SemaphoreType.DMA,
            pltpu.SemaphoreType.DMA,
            pltpu.SemaphoreType.DMA,
            pltpu.SemaphoreType.DMA,
        ],
    )
    def run(in_hbm, z_hbm, out_hbm, b0, b1, b2, b3, b4, b5,
            i0, i1, i2, i3, i4, i5, o0, o1, o2, o3, o4, o5):
        wid = lax.axis_index("s") * info.num_cores + lax.axis_index("c")
        bw = wid // wpb
        h = wid % wpb
        t0 = h * tn
        is_drop = functools.reduce(
            jnp.logical_or, [bw == i for i in dropped], jnp.bool_(False)
        )
        is_last = h == wpb - 1
        bufs = (b0, b1, b2, b3, b4, b5)
        isems = (i0, i1, i2, i3, i4, i5)
        osems = (o0, o1, o2, o3, o4, o5)
        nring = len(bufs)

        def src(i):
            return in_hbm.at[bw, pl.ds(t0 + i * ch, ch)]

        def dst(i):
            return out_hbm.at[bw, pl.ds(t0 + i * ch, ch)]

        @pl.when(jnp.logical_not(is_drop))
        def _copy():
            in_d = [None] * nch
            out_d = [None] * nch
            in_d[0] = pltpu.async_copy(src(0), bufs[0], isems[0])
            for i in range(nch):
                p = i % nring
                if i + 1 < nch:
                    q = (i + 1) % nring
                    if i + 1 >= nring:
                        out_d[i + 1 - nring].wait()
                    in_d[i + 1] = pltpu.async_copy(src(i + 1), bufs[q], isems[q])
                in_d[i].wait()
                out_d[i] = pltpu.async_copy(bufs[p], dst(i), osems[p])
            for j in range(max(0, nch - nring), nch):
                out_d[j].wait()

        @pl.when(is_drop)
        def _zero():
            pltpu.async_copy(z_hbm.at[pl.ds(0, ch)], b0, i0).wait()
            out_d = [pltpu.async_copy(b0, dst(i), o0) for i in range(nch - 1)]
            for d_ in out_d:
                d_.wait()

            @pl.when(jnp.logical_not(is_last))
            def _full_tail():
                pltpu.async_copy(b0, dst(nch - 1), o0).wait()

            @pl.when(is_last)
            def _partial_tail():
                pltpu.async_copy(
                    b0.at[pl.ds(0, ch - 1)],
                    out_hbm.at[bw, pl.ds(t0 + (nch - 1) * ch, ch - 1)],
                    o0,
                ).wait()
                pltpu.async_copy(
                    in_hbm.at[bw, pl.ds(t - 1, 1)], b1.at[pl.ds(0, 1)], i1
                ).wait()
                pltpu.async_copy(
                    b1.at[pl.ds(0, 1)], out_hbm.at[bw, pl.ds(t - 1, 1)], o1
                ).wait()

    return run(emb0, zeros)


def _tc_part(emb0, s):
    b, t, c, d = emb0.shape

    def body(in_ref, out_ref):
        out_ref[...] = in_ref[...]

    return pl.pallas_call(
        body,
        grid=(b - s,),
        in_specs=[pl.BlockSpec((1, t, c, d), lambda i: (i + s, 0, 0, 0))],
        out_specs=pl.BlockSpec((1, t, c, d), lambda i: (i, 0, 0, 0)),
        out_shape=jax.ShapeDtypeStruct((b - s, t, c, d), emb0.dtype),
    )(emb0)


def kernel(emb0):
    b, t, c, d = emb0.shape
    dropped = set(_dropped_ids(b))
    s = b // 2
    if not dropped or max(dropped) >= s or (b * t * c * d) % (32 * s) != 0:
        s = b
    sc_out = _sc_part(emb0, s, dropped)
    if s == b:
        return sc_out
    tc_out = _tc_part(emb0, s)
    return jnp.concatenate([sc_out, tc_out], axis=0)


# speedup vs baseline: 1.5966x; 1.5966x over previous
import functools

import numpy as np
import jax
import jax.numpy as jnp
from jax import lax
from jax.experimental import pallas as pl
from jax.experimental.pallas import tpu as pltpu, tpu_sc as plsc

_PROB = 0.25

_DROPPED_B16 = (7, 6, 3, 2)


@functools.lru_cache(maxsize=None)
def _dropped_ids(b):
    num = 1 if b == 1 else int(b * _PROB)
    if b == 16:
        return _DROPPED_B16[:num]
    with jax.ensure_compile_time_eval(), jax.default_device(jax.devices("cpu")[0]):
        perm = np.asarray(jax.random.permutation(jax.random.key(1), b))
    return tuple(int(x) for x in perm[:num])


def _sc_part(emb0, s, dropped):
    b, t, c, d = emb0.shape
    info = plsc.get_sparse_core_info()
    nw = info.num_cores * info.num_subcores
    wpb = nw // s
    tn = t // wpb
    ch = 32
    nch = tn // ch
    mesh = plsc.VectorSubcoreMesh(core_axis_name="c", subcore_axis_name="s")

    zeros = jnp.zeros((ch, c, d), emb0.dtype)

    @functools.partial(
        pl.kernel,
        out_type=jax.ShapeDtypeStruct((b, t, c, d), emb0.dtype),
        mesh=mesh,
        scratch_types=[
            pltpu.VMEM((ch, c, d), emb0.dtype),
            pltpu.VMEM((ch, c, d), emb0.dtype),
            pltpu.VMEM((ch, c, d), emb0.dtype),
            pltpu.VMEM((ch, c, d), emb0.dtype),
            pltpu.VMEM((ch, c, d), emb0.dtype),
            pltpu.VMEM((ch, c, d), emb0.dtype),
            pltpu.SemaphoreType.DMA,
            pltpu.SemaphoreType.DMA,
            pltpu.SemaphoreType.DMA,
            pltpu.SemaphoreType.DMA,
            pltpu.SemaphoreType.DMA,
            pltpu.SemaphoreType.DMA,
            pltpu.SemaphoreType.DMA,
            pltpu.SemaphoreType.DMA,
            pltpu.SemaphoreType.DMA,
            pltpu.SemaphoreType.DMA,
            pltpu.SemaphoreType.DMA,
            pltpu.SemaphoreType.DMA,
        ],
    )
    def run(in_hbm, z_hbm, out_hbm, b0, b1, b2, b3, b4, b5,
            i0, i1, i2, i3, i4, i5, o0, o1, o2, o3, o4, o5):
        wid = lax.axis_index("s") * info.num_cores + lax.axis_index("c")
        bw = wid // wpb
        h = wid % wpb
        t0 = h * tn
        is_drop = functools.reduce(
            jnp.logical_or, [bw == i for i in dropped], jnp.bool_(False)
        )
        is_last = h == wpb - 1
        bufs = (b0, b1, b2, b3, b4, b5)
        isems = (i0, i1, i2, i3, i4, i5)
        osems = (o0, o1, o2, o3, o4, o5)
        nring = len(bufs)

        def src(i):
            return in_hbm.at[bw, pl.ds(t0 + i * ch, ch)]

        def dst(i):
            return out_hbm.at[bw, pl.ds(t0 + i * ch, ch)]

        @pl.when(jnp.logical_not(is_drop))
        def _copy():
            in_d = [None] * nch
            out_d = [None] * nch
            in_d[0] = pltpu.async_copy(src(0), bufs[0], isems[0])
            for i in range(nch):
                p = i % nring
                if i + 1 < nch:
                    q = (i + 1) % nring
                    if i + 1 >= nring:
                        out_d[i + 1 - nring].wait()
                    in_d[i + 1] = pltpu.async_copy(src(i + 1), bufs[q], isems[q])
                in_d[i].wait()
                out_d[i] = pltpu.async_copy(bufs[p], dst(i), osems[p])
            for j in range(max(0, nch - nring), nch):
                out_d[j].wait()

        @pl.when(is_drop)
        def _zero():
            pltpu.async_copy(z_hbm.at[pl.ds(0, ch)], b0, i0).wait()
            out_d = [pltpu.async_copy(b0, dst(i), o0) for i in range(nch - 1)]
            for d_ in out_d:
                d_.wait()

            @pl.when(jnp.logical_not(is_last))
            def _full_tail():
                pltpu.async_copy(b0, dst(nch - 1), o0).wait()

            @pl.when(is_last)
            def _partial_tail():
                pltpu.async_copy(
                    b0.at[pl.ds(0, ch - 1)],
                    out_hbm.at[bw, pl.ds(t0 + (nch - 1) * ch, ch - 1)],
                    o0,
                ).wait()
                pltpu.async_copy(
                    in_hbm.at[bw, pl.ds(t - 1, 1)], b1.at[pl.ds(0, 1)], i1
                ).wait()
                pltpu.async_copy(
                    b1.at[pl.ds(0, 1)], out_hbm.at[bw, pl.ds(t - 1, 1)], o1
                ).wait()

    return run(emb0, zeros)


def _tc_fill(partial_out, emb0, s):
    b, t, c, d = emb0.shape

    def body(acc_ref, in_ref, out_ref):
        del acc_ref
        out_ref[...] = in_ref[...]

    return pl.pallas_call(
        body,
        grid=(b - s,),
        in_specs=[
            pl.BlockSpec(memory_space=pltpu.MemorySpace.HBM),
            pl.BlockSpec((1, t, c, d), lambda i: (i + s, 0, 0, 0)),
        ],
        out_specs=pl.BlockSpec((1, t, c, d), lambda i: (i + s, 0, 0, 0)),
        out_shape=jax.ShapeDtypeStruct((b, t, c, d), emb0.dtype),
        input_output_aliases={0: 0},
    )(partial_out, emb0)


def kernel(emb0):
    b, t, c, d = emb0.shape
    dropped = set(_dropped_ids(b))
    s = b // 2
    if not dropped or max(dropped) >= s or (b * t * c * d) % (32 * s) != 0:
        s = b
    sc_out = _sc_part(emb0, s, dropped)
    if s == b:
        return sc_out
    return _tc_fill(sc_out, emb0, s)
